# baseline (speedup 1.0000x reference)
import functools
import math

import jax
import jax.numpy as jnp
from jax import lax
from jax.experimental import pallas as pl
from jax.experimental.pallas import tpu as pltpu
from jax.experimental.pallas import tpu_sc as plsc

_SMOOTHING = 0.1


def _sc_main(output, labels_flat, S):
    B, V = output.shape
    info = plsc.get_sparse_core_info()
    NC, NS = info.num_cores, info.num_subcores
    NW = NC * NS
    RPT = B // NW
    SP = labels_flat.shape[0] // B
    NL = RPT * SP
    P = 50000
    NPC = V // P
    NP = RPT * NPC
    UN = 125
    NACC = 8

    mesh = plsc.VectorSubcoreMesh(core_axis_name="c", subcore_axis_name="s")

    @functools.partial(
        pl.kernel,
        mesh=mesh,
        out_type=(
            jax.ShapeDtypeStruct((NW * 16,), jnp.float32),
            jax.ShapeDtypeStruct((B * SP,), jnp.float32),
        ),
        compiler_params=pltpu.CompilerParams(use_tc_tiling_on_sc=False),
        scratch_types=[
            pltpu.VMEM((NL,), jnp.int32),
            pltpu.VMEM((NL,), jnp.float32),
            pltpu.VMEM((2 * P,), jnp.float32),
            pltpu.SemaphoreType.DMA,
            pltpu.SemaphoreType.DMA((2,)),
        ],
    )
    def skern(out_hbm, lab_hbm, part_hbm, vals_hbm, lab_v, vals_v, buf_v,
              gsem, sems):
        wid = lax.axis_index("s") * NC + lax.axis_index("c")
        base = wid * RPT

        pltpu.sync_copy(lab_hbm.at[pl.ds(base * SP, NL)], lab_v)
        for r in range(RPT):
            pltpu.async_copy(
                out_hbm.at[base + r].at[lab_v.at[pl.ds(r * SP, S)]],
                vals_v.at[pl.ds(r * SP, S)],
                gsem,
            )

        def piece_src(p):
            return out_hbm.at[base + p // NPC].at[pl.ds((p % NPC) * P, P)]

        pltpu.async_copy(piece_src(0), buf_v.at[pl.ds(0, P)], sems.at[0])

        def step(p, total):
            par = lax.rem(p, 2)
            nxt = lax.rem(p + 1, 2)
            pltpu.async_copy(
                piece_src(lax.rem(p + 1, NP)),
                buf_v.at[pl.ds(nxt * P, P)],
                sems.at[nxt],
            )
            pltpu.make_async_copy(
                piece_src(0), buf_v.at[pl.ds(0, P)], sems.at[par]
            ).wait()
            off0 = par * P
            zero = jnp.zeros((16,), jnp.float32)

            def inner(i, accs):
                o = off0 + i * (UN * 16)
                accs = list(accs)
                for j in range(UN):
                    accs[j % NACC] = accs[j % NACC] + buf_v[
                        pl.ds(o + j * 16, 16)
                    ]
                return tuple(accs)

            accs = lax.fori_loop(0, P // (UN * 16), inner, (zero,) * NACC)
            piece_total = accs[0]
            for a in accs[1:]:
                piece_total = piece_total + a
            return total + piece_total

        total = lax.fori_loop(0, NP, step, jnp.zeros((16,), jnp.float32))
        pltpu.make_async_copy(
            piece_src(0), buf_v.at[pl.ds(0, P)], sems.at[lax.rem(NP, 2)]
        ).wait()
        buf_v[pl.ds(0, 16)] = total
        pltpu.sync_copy(
            buf_v.at[pl.ds(0, 16)], part_hbm.at[pl.ds(wid * 16, 16)]
        )

        for r in range(RPT):
            pltpu.make_async_copy(
                out_hbm.at[0].at[lab_v.at[pl.ds(0, S)]],
                vals_v.at[pl.ds(0, S)],
                gsem,
            ).wait()
        pltpu.sync_copy(vals_v, vals_hbm.at[pl.ds(base * SP, NL)])

    return skern(output, labels_flat)


def _tc_sum(output, rows):
    B, V = output.shape
    BR = 32
    grid = rows // BR

    def body(out_ref, acc_ref):
        @pl.when(pl.program_id(0) == 0)
        def _init():
            acc_ref[0, 0] = jnp.float32(0.0)

        acc_ref[0, 0] += jnp.sum(out_ref[...])

    return pl.pallas_call(
        body,
        grid=(grid,),
        in_specs=[pl.BlockSpec((BR, V), lambda g: (g, 0))],
        out_specs=pl.BlockSpec(
            (1, 1), lambda g: (0, 0), memory_space=pltpu.SMEM
        ),
        out_shape=jax.ShapeDtypeStruct((1, 1), jnp.float32),
    )(output)


def _tc_combine(T, vals, batch_labels, B, V):
    S = batch_labels.shape[1]
    fill = _SMOOTHING / (V - S)
    lab = (1.0 - _SMOOTHING) / S
    c_base = B * V * fill * math.log(fill)
    c_per_d = lab * math.log(lab) - fill * math.log(fill)

    def body(t_ref, vals_ref, lab_ref, loss_ref):
        T = jnp.sum(t_ref[...])
        labels = lab_ref[...]
        col = lax.broadcasted_iota(jnp.int32, labels.shape, 1)
        dup = jnp.zeros(labels.shape, dtype=jnp.bool_)
        for s in range(1, S):
            sh = jnp.concatenate([labels[:, :s], labels[:, : S - s]], axis=1)
            dup = jnp.logical_or(dup, jnp.logical_and(labels == sh, col >= s))
        m = jnp.where(dup, jnp.float32(0.0), jnp.float32(1.0))
        D = jnp.sum(m)
        G = jnp.sum(m * vals_ref[...])
        loss_ref[0, 0] = (
            jnp.float32(c_base)
            + D * jnp.float32(c_per_d)
            - jnp.float32(fill) * T
            - jnp.float32(lab - fill) * G
        )

    return pl.pallas_call(
        body,
        in_specs=[
            pl.BlockSpec(T.shape, lambda: (0,) * len(T.shape)),
            pl.BlockSpec((B, S), lambda: (0, 0)),
            pl.BlockSpec((B, S), lambda: (0, 0)),
        ],
        out_specs=pl.BlockSpec(memory_space=pltpu.SMEM),
        out_shape=jax.ShapeDtypeStruct((1, 1), jnp.float32),
    )(T, vals, batch_labels)


def kernel(output, batch_labels):
    B, V = output.shape
    S = batch_labels.shape[1]
    SP = (S + 7) // 8 * 8
    labels_pad = jnp.pad(batch_labels, ((0, 0), (0, SP - S)))
    parts, vals = _sc_main(output, labels_pad.reshape(B * SP), S)
    loss = _tc_combine(
        parts.reshape(1, -1), vals.reshape(B, SP)[:, :S], batch_labels, B, V
    )
    return loss[0, 0]


# speedup vs baseline: 1.0078x; 1.0078x over previous
import functools
import math

import jax
import jax.numpy as jnp
from jax import lax
from jax.experimental import pallas as pl
from jax.experimental.pallas import tpu as pltpu
from jax.experimental.pallas import tpu_sc as plsc

_SMOOTHING = 0.1


def _sc_main(output, labels_flat, S):
    B, V = output.shape
    info = plsc.get_sparse_core_info()
    NC, NS = info.num_cores, info.num_subcores
    NW = NC * NS
    RPT = B // NW
    SP = labels_flat.shape[0] // B
    NL = RPT * SP
    P = 50000
    NPC = V // P
    NP = RPT * NPC
    UN = 25
    NACC = 8

    mesh = plsc.VectorSubcoreMesh(core_axis_name="c", subcore_axis_name="s")

    @functools.partial(
        pl.kernel,
        mesh=mesh,
        out_type=(
            jax.ShapeDtypeStruct((NW * 16,), jnp.float32),
            jax.ShapeDtypeStruct((B * SP,), jnp.float32),
        ),
        compiler_params=pltpu.CompilerParams(use_tc_tiling_on_sc=False),
        scratch_types=[
            pltpu.VMEM((NL,), jnp.int32),
            pltpu.VMEM((NL,), jnp.float32),
            pltpu.VMEM((2 * P,), jnp.float32),
            pltpu.SemaphoreType.DMA,
            pltpu.SemaphoreType.DMA((2,)),
        ],
    )
    def skern(out_hbm, lab_hbm, part_hbm, vals_hbm, lab_v, vals_v, buf_v,
              gsem, sems):
        wid = lax.axis_index("s") * NC + lax.axis_index("c")
        base = wid * RPT

        pltpu.sync_copy(lab_hbm.at[pl.ds(base * SP, NL)], lab_v)
        for r in range(RPT):
            pltpu.async_copy(
                out_hbm.at[base + r].at[lab_v.at[pl.ds(r * SP, S)]],
                vals_v.at[pl.ds(r * SP, S)],
                gsem,
            )

        def piece_src(p):
            return out_hbm.at[base + p // NPC].at[pl.ds((p % NPC) * P, P)]

        pltpu.async_copy(piece_src(0), buf_v.at[pl.ds(0, P)], sems.at[0])

        def step(p, total):
            par = lax.rem(p, 2)
            nxt = lax.rem(p + 1, 2)
            pltpu.async_copy(
                piece_src(lax.rem(p + 1, NP)),
                buf_v.at[pl.ds(nxt * P, P)],
                sems.at[nxt],
            )
            pltpu.make_async_copy(
                piece_src(0), buf_v.at[pl.ds(0, P)], sems.at[par]
            ).wait()
            off0 = par * P
            zero = jnp.zeros((16,), jnp.float32)

            def inner(i, accs):
                o = off0 + i * (UN * 16)
                accs = list(accs)
                for j in range(UN):
                    accs[j % NACC] = accs[j % NACC] + buf_v[
                        pl.ds(o + j * 16, 16)
                    ]
                return tuple(accs)

            accs = lax.fori_loop(0, P // (UN * 16), inner, (zero,) * NACC)
            piece_total = accs[0]
            for a in accs[1:]:
                piece_total = piece_total + a
            return total + piece_total

        total = lax.fori_loop(0, NP, step, jnp.zeros((16,), jnp.float32))
        pltpu.make_async_copy(
            piece_src(0), buf_v.at[pl.ds(0, P)], sems.at[lax.rem(NP, 2)]
        ).wait()
        buf_v[pl.ds(0, 16)] = total
        pltpu.sync_copy(
            buf_v.at[pl.ds(0, 16)], part_hbm.at[pl.ds(wid * 16, 16)]
        )

        for r in range(RPT):
            pltpu.make_async_copy(
                out_hbm.at[0].at[lab_v.at[pl.ds(0, S)]],
                vals_v.at[pl.ds(0, S)],
                gsem,
            ).wait()
        pltpu.sync_copy(vals_v, vals_hbm.at[pl.ds(base * SP, NL)])

    return skern(output, labels_flat)


def _tc_sum(output, rows):
    B, V = output.shape
    BR = 32
    grid = rows // BR

    def body(out_ref, acc_ref):
        @pl.when(pl.program_id(0) == 0)
        def _init():
            acc_ref[0, 0] = jnp.float32(0.0)

        acc_ref[0, 0] += jnp.sum(out_ref[...])

    return pl.pallas_call(
        body,
        grid=(grid,),
        in_specs=[pl.BlockSpec((BR, V), lambda g: (g, 0))],
        out_specs=pl.BlockSpec(
            (1, 1), lambda g: (0, 0), memory_space=pltpu.SMEM
        ),
        out_shape=jax.ShapeDtypeStruct((1, 1), jnp.float32),
    )(output)


def _tc_combine(T, vals, batch_labels, B, V):
    S = batch_labels.shape[1]
    fill = _SMOOTHING / (V - S)
    lab = (1.0 - _SMOOTHING) / S
    c_base = B * V * fill * math.log(fill)
    c_per_d = lab * math.log(lab) - fill * math.log(fill)

    def body(t_ref, vals_ref, lab_ref, loss_ref):
        T = jnp.sum(t_ref[...])
        labels = lab_ref[...]
        col = lax.broadcasted_iota(jnp.int32, labels.shape, 1)
        dup = jnp.zeros(labels.shape, dtype=jnp.bool_)
        for s in range(1, S):
            sh = jnp.concatenate([labels[:, :s], labels[:, : S - s]], axis=1)
            dup = jnp.logical_or(dup, jnp.logical_and(labels == sh, col >= s))
        m = jnp.where(dup, jnp.float32(0.0), jnp.float32(1.0))
        D = jnp.sum(m)
        G = jnp.sum(m * vals_ref[...])
        loss_ref[0, 0] = (
            jnp.float32(c_base)
            + D * jnp.float32(c_per_d)
            - jnp.float32(fill) * T
            - jnp.float32(lab - fill) * G
        )

    return pl.pallas_call(
        body,
        in_specs=[
            pl.BlockSpec(T.shape, lambda: (0,) * len(T.shape)),
            pl.BlockSpec((B, S), lambda: (0, 0)),
            pl.BlockSpec((B, S), lambda: (0, 0)),
        ],
        out_specs=pl.BlockSpec(memory_space=pltpu.SMEM),
        out_shape=jax.ShapeDtypeStruct((1, 1), jnp.float32),
    )(T, vals, batch_labels)


def kernel(output, batch_labels):
    B, V = output.shape
    S = batch_labels.shape[1]
    SP = (S + 7) // 8 * 8
    labels_pad = jnp.pad(batch_labels, ((0, 0), (0, SP - S)))
    parts, vals = _sc_main(output, labels_pad.reshape(B * SP), S)
    loss = _tc_combine(
        parts.reshape(1, -1), vals.reshape(B, SP)[:, :S], batch_labels, B, V
    )
    return loss[0, 0]
